# Initial kernel scaffold; baseline (speedup 1.0000x reference)
import jax
import jax.numpy as jnp
from jax.experimental import pallas as pl


def kernel(edge_index, edge_weight, user_emb, item_emb, W1, b1, W2, b2):
    raise NotImplementedError("write your pallas kernel here")


# speedup vs baseline: 3.6109x; 3.6109x over previous
import functools

import jax
import jax.numpy as jnp
from jax import lax
from jax.experimental import pallas as pl
from jax.experimental.pallas import tpu as pltpu
from jax.experimental.pallas import tpu_sc as plsc

NUM_USERS = 20000
NUM_ITEMS = 30000
N_NODES = NUM_USERS + NUM_ITEMS
D = 64
E = 800000

NC = 2
NS = 16
L = 16

HALF = N_NODES // NC
ROWS_PER_TILE = 1664
ACC_ROWS = NS * ROWS_PER_TILE
DUMP = 25600

SB = 256
G = 128
NG = SB // G
E_PAD = 802816
EPT = E_PAD // NS
NSC = EPT // SB

_mesh = plsc.VectorSubcoreMesh(core_axis_name="c", subcore_axis_name="s")


@functools.partial(
    pl.kernel,
    mesh=_mesh,
    out_type=jax.ShapeDtypeStruct((N_NODES, D), jnp.float32),
    compiler_params=pltpu.CompilerParams(use_tc_tiling_on_sc=False),
    scratch_types=[
        pltpu.VMEM((SB,), jnp.int32),
        pltpu.VMEM((SB,), jnp.int32),
        pltpu.VMEM((NG, G), jnp.int32),
        pltpu.VMEM((SB,), jnp.float32),
        pltpu.VMEM((SB, D), jnp.float32),
        pltpu.VMEM_SHARED((ACC_ROWS, D), jnp.float32),
        pltpu.SemaphoreType.DMA,
    ],
)
def _sc_agg(src_hbm, dst_hbm, w_hbm, x_hbm, out_hbm,
            src_v, dst_v, adj_v, w_v, rows_v, acc, sem):
    cid = lax.axis_index("c")
    sid = lax.axis_index("s")
    lo = cid * HALF

    def _zrow(i, carry):
        for j in range(D // L):
            rows_v[i, pl.ds(j * L, L)] = jnp.zeros((L,), jnp.float32)
        return carry
    lax.fori_loop(0, G, _zrow, None)

    def _zcp(k, carry):
        r = sid * ROWS_PER_TILE + k * G
        pltpu.sync_copy(rows_v.at[pl.ds(0, G)], acc.at[pl.ds(r, G)])
        return carry
    lax.fori_loop(0, ROWS_PER_TILE // G, _zcp, None)
    plsc.subcore_barrier()

    def _chunk(ci, carry):
        e0 = sid * EPT + ci * SB
        pltpu.sync_copy(src_hbm.at[pl.ds(e0, SB)], src_v)
        pltpu.sync_copy(dst_hbm.at[pl.ds(e0, SB)], dst_v)
        pltpu.sync_copy(w_hbm.at[pl.ds(e0, SB)], w_v)

        for v in range(SB // L):
            dvec = dst_v[pl.ds(v * L, L)]
            m = (dvec >= lo) & (dvec < lo + HALF)
            adj_v[v // (G // L), pl.ds((v % (G // L)) * L, L)] = (
                jnp.where(m, dvec - lo, DUMP))

        copies = [
            pltpu.async_copy(
                x_hbm.at[src_v.at[pl.ds(g * G, G)]],
                rows_v.at[pl.ds(g * G, G)], sem)
            for g in range(NG)
        ]
        for c in copies:
            c.wait()

        def _srow(t, carry2):
            i0 = t * L
            wvec = w_v[pl.ds(i0, L)]
            for k in range(L):
                w = wvec[k]
                for j in range(D // L):
                    rows_v[i0 + k, pl.ds(j * L, L)] = (
                        rows_v[i0 + k, pl.ds(j * L, L)] * w)
            return carry2
        lax.fori_loop(0, SB // L, _srow, None)

        for g in range(NG):
            pltpu.sync_copy(rows_v.at[pl.ds(g * G, G)],
                            acc.at[adj_v.at[g]], add=True)
        return carry
    lax.fori_loop(0, NSC, _chunk, None)

    plsc.subcore_barrier()

    nfull = jnp.where(sid < NS - 1, ROWS_PER_TILE // G, 0)

    def _wcp(k, carry):
        r = sid * ROWS_PER_TILE + k * G
        pltpu.sync_copy(acc.at[pl.ds(r, G)], out_hbm.at[pl.ds(lo + r, G)])
        return carry
    lax.fori_loop(0, nfull, _wcp, None)

    @pl.when(sid == NS - 1)
    def _tail():
        r = (NS - 1) * ROWS_PER_TILE
        pltpu.sync_copy(acc.at[pl.ds(r, HALF - r)],
                        out_hbm.at[pl.ds(lo + r, HALF - r)])


ROWS_BLK = 1000


def _dense_body(agg_ref, x_ref, wt_ref, b_ref, o_ref):
    xb = (agg_ref[...] + x_ref[...]) * 0.5
    o_ref[...] = (jnp.dot(xb, wt_ref[...], preferred_element_type=jnp.float32)
                  + b_ref[...])


def _dense(agg, x, Wt, b2d):
    return pl.pallas_call(
        _dense_body,
        grid=(N_NODES // ROWS_BLK,),
        in_specs=[
            pl.BlockSpec((ROWS_BLK, D), lambda i: (i, 0)),
            pl.BlockSpec((ROWS_BLK, D), lambda i: (i, 0)),
            pl.BlockSpec((D, D), lambda i: (0, 0)),
            pl.BlockSpec((1, D), lambda i: (0, 0)),
        ],
        out_specs=pl.BlockSpec((ROWS_BLK, D), lambda i: (i, 0)),
        out_shape=jax.ShapeDtypeStruct((N_NODES, D), jnp.float32),
    )(agg, x, Wt, b2d)


def kernel(edge_index, edge_weight, user_emb, item_emb, W1, b1, W2, b2):
    x = jnp.concatenate([user_emb, item_emb], axis=0)
    src = edge_index[0].astype(jnp.int32)
    dst = edge_index[1].astype(jnp.int32)
    w = edge_weight.astype(jnp.float32)
    pad = E_PAD - E
    src = jnp.concatenate([src, jnp.zeros((pad,), jnp.int32)])
    dst = jnp.concatenate([dst, jnp.zeros((pad,), jnp.int32)])
    w = jnp.concatenate([w, jnp.zeros((pad,), jnp.float32)])

    wt1, wt2 = W1.T, W2.T
    b1r, b2r = b1.reshape(1, D), b2.reshape(1, D)

    agg1 = _sc_agg(src, dst, w, x)
    h1 = _dense(agg1, x, wt1, b1r)
    agg2 = _sc_agg(src, dst, w, h1)
    return _dense(agg2, h1, wt2, b2r)
